# Initial kernel scaffold; baseline (speedup 1.0000x reference)
import jax
import jax.numpy as jnp
from jax.experimental import pallas as pl


def kernel(means3D, means2D, opacities, colors_precomp, scales, rotations, bg, viewmatrix, projmatrix, campos):
    raise NotImplementedError("write your pallas kernel here")


# speedup vs baseline: 1.6890x; 1.6890x over previous
import functools

import jax
import jax.numpy as jnp
from jax import lax
from jax.experimental import pallas as pl
from jax.experimental.pallas import tpu as pltpu

P = 2048
H = 64
W = 64
HW = H * W
TANFOVX = 0.5773502691896257
TANFOVY = 0.5773502691896257
SCALE_MODIFIER = 1.0
FOCAL_X = W / (2.0 * TANFOVX)
FOCAL_Y = H / (2.0 * TANFOVY)
LIMX = 1.3 * TANFOVX
LIMY = 1.3 * TANFOVY

CH = 256
NPC = 256
NROWS = 16


def _dot_t(a, b):
    return lax.dot_general(a, b, (((1,), (1,)), ((), ())),
                           preferred_element_type=jnp.float32,
                           precision=lax.Precision.HIGHEST)


def _bf(x):
    return x.astype(jnp.bfloat16).astype(jnp.float32)


def _prep_kernel(g3_ref, opr_ref, colr_ref, scr_ref, rotr_ref, vm_ref, pm_ref,
                 rows_ref, cols_ref, radii_ref):
    x = g3_ref[0:1, :]
    y = g3_ref[1:2, :]
    z = g3_ref[2:3, :]

    def vm(i, j):
        return vm_ref[i, j]

    def pm(i, j):
        return pm_ref[i, j]

    bx, by, bz = _bf(x), _bf(y), _bf(z)
    tx = bx * vm(0, 0) + by * vm(0, 1) + bz * vm(0, 2) + vm(0, 3)
    ty = bx * vm(1, 0) + by * vm(1, 1) + bz * vm(1, 2) + vm(1, 3)
    tz = bx * vm(2, 0) + by * vm(2, 1) + bz * vm(2, 2) + vm(2, 3)

    hx = bx * pm(0, 0) + by * pm(0, 1) + bz * pm(0, 2) + pm(0, 3)
    hy = bx * pm(1, 0) + by * pm(1, 1) + bz * pm(1, 2) + pm(1, 3)
    hw = bx * pm(3, 0) + by * pm(3, 1) + bz * pm(3, 2) + pm(3, 3)
    wdiv = hw + 1e-7
    px = ((hx / wdiv + 1.0) * W - 1.0) * 0.5
    py = ((hy / wdiv + 1.0) * H - 1.0) * 0.5

    qr = rotr_ref[0:1, :]
    qx = rotr_ref[1:2, :]
    qy = rotr_ref[2:3, :]
    qz = rotr_ref[3:4, :]
    qn = jnp.sqrt(qr * qr + qx * qx + qy * qy + qz * qz) + 1e-12
    qr = qr / qn
    qx = qx / qn
    qy = qy / qn
    qz = qz / qn
    r00 = 1 - 2 * (qy * qy + qz * qz)
    r01 = 2 * (qx * qy - qr * qz)
    r02 = 2 * (qx * qz + qr * qy)
    r10 = 2 * (qx * qy + qr * qz)
    r11 = 1 - 2 * (qx * qx + qz * qz)
    r12 = 2 * (qy * qz - qr * qx)
    r20 = 2 * (qx * qz - qr * qy)
    r21 = 2 * (qy * qz + qr * qx)
    r22 = 1 - 2 * (qx * qx + qy * qy)

    sx = scr_ref[0:1, :] * SCALE_MODIFIER
    sy = scr_ref[1:2, :] * SCALE_MODIFIER
    sz = scr_ref[2:3, :] * SCALE_MODIFIER
    m00, m01, m02 = _bf(r00 * sx), _bf(r01 * sy), _bf(r02 * sz)
    m10, m11, m12 = _bf(r10 * sx), _bf(r11 * sy), _bf(r12 * sz)
    m20, m21, m22 = _bf(r20 * sx), _bf(r21 * sy), _bf(r22 * sz)
    v00 = m00 * m00 + m01 * m01 + m02 * m02
    v01 = m00 * m10 + m01 * m11 + m02 * m12
    v02 = m00 * m20 + m01 * m21 + m02 * m22
    v11 = m10 * m10 + m11 * m11 + m12 * m12
    v12 = m10 * m20 + m11 * m21 + m12 * m22
    v22 = m20 * m20 + m21 * m21 + m22 * m22

    tz_safe = jnp.where(jnp.abs(tz) > 1e-6, tz, 1e-6)
    txc = jnp.clip(tx / tz_safe, -LIMX, LIMX) * tz_safe
    tyc = jnp.clip(ty / tz_safe, -LIMY, LIMY) * tz_safe
    inv_tz = 1.0 / tz_safe
    j00 = _bf(FOCAL_X * inv_tz)
    j02 = _bf(-FOCAL_X * txc * inv_tz * inv_tz)
    j11 = _bf(FOCAL_Y * inv_tz)
    j12 = _bf(-FOCAL_Y * tyc * inv_tz * inv_tz)
    t00 = j00 * vm(0, 0) + j02 * vm(2, 0)
    t01 = j00 * vm(0, 1) + j02 * vm(2, 1)
    t02 = j00 * vm(0, 2) + j02 * vm(2, 2)
    t10 = j11 * vm(1, 0) + j12 * vm(2, 0)
    t11 = j11 * vm(1, 1) + j12 * vm(2, 1)
    t12 = j11 * vm(1, 2) + j12 * vm(2, 2)
    bt00, bt01, bt02 = _bf(t00), _bf(t01), _bf(t02)
    bt10, bt11, bt12 = _bf(t10), _bf(t11), _bf(t12)
    bv00, bv01, bv02 = _bf(v00), _bf(v01), _bf(v02)
    bv11, bv12, bv22 = _bf(v11), _bf(v12), _bf(v22)
    u00 = bt00 * bv00 + bt01 * bv01 + bt02 * bv02
    u01 = bt00 * bv01 + bt01 * bv11 + bt02 * bv12
    u02 = bt00 * bv02 + bt01 * bv12 + bt02 * bv22
    u10 = bt10 * bv00 + bt11 * bv01 + bt12 * bv02
    u11 = bt10 * bv01 + bt11 * bv11 + bt12 * bv12
    u12 = bt10 * bv02 + bt11 * bv12 + bt12 * bv22
    bu00, bu01, bu02 = _bf(u00), _bf(u01), _bf(u02)
    bu10, bu11, bu12 = _bf(u10), _bf(u11), _bf(u12)
    c00 = bu00 * bt00 + bu01 * bt01 + bu02 * bt02 + 0.3
    c01 = bu00 * bt10 + bu01 * bt11 + bu02 * bt12
    c11 = bu10 * bt10 + bu11 * bt11 + bu12 * bt12 + 0.3

    det = c00 * c11 - c01 * c01
    det_safe = jnp.where(det != 0.0, det, 1.0)
    con_a = c11 / det_safe
    con_b = -c01 / det_safe
    con_c = c00 / det_safe
    mid = 0.5 * (c00 + c11)
    lam1 = mid + jnp.sqrt(jnp.maximum(0.1, mid * mid - det))
    valid = (det > 0.0) & (tz > 0.2)
    radii = jnp.where(valid, jnp.ceil(3.0 * jnp.sqrt(lam1)), 0.0)
    radii_ref[0:1, :] = radii.astype(jnp.int32)
    validf = valid.astype(jnp.float32)

    lane_i = lax.broadcasted_iota(jnp.int32, (1, P), 1)
    rank = jnp.zeros((1, P), jnp.float32)
    for c in range(P // CH):
        off = c * CH
        rowi = lax.broadcasted_iota(jnp.int32, (CH, P), 0) + off
        lane2 = lax.broadcasted_iota(jnp.int32, (CH, P), 1)
        eye_chunk = (rowi == lane2).astype(jnp.float32)
        zc = _dot_t(eye_chunk, tz)
        jcol = lax.broadcasted_iota(jnp.int32, (CH, 1), 0) + off
        before = (zc < tz) | ((zc == tz) & (jcol < lane_i))
        rank = rank + jnp.sum(before.astype(jnp.float32), axis=0, keepdims=True)

    op_row = opr_ref[0:1, :]
    data = jnp.concatenate([
        px, py, con_a, con_b, con_c, op_row,
        colr_ref[0:1, :], colr_ref[1:2, :], colr_ref[2:3, :],
        inv_tz, validf,
        jnp.zeros((NROWS - 11, P), jnp.float32),
    ], axis=0)

    for c in range(P // CH):
        off = c * CH
        srow = (lax.broadcasted_iota(jnp.int32, (CH, P), 0) + off).astype(jnp.float32)
        oh_t = (rank == srow).astype(jnp.float32)
        rows_ref[:, off:off + CH] = _dot_t(data, oh_t)
        cols_ref[off:off + CH, :] = _dot_t(oh_t, data)


def _comp_kernel(rows_ref, cols_ref, bg_ref, out_ref):
    pid = pl.program_id(0)
    base = pid * NPC
    pidx = lax.broadcasted_iota(jnp.int32, (NPC, 1), 0) + base
    xf = (pidx & (W - 1)).astype(jnp.float32)
    yf = (pidx >> 6).astype(jnp.float32)

    px = rows_ref[0:1, :]
    py = rows_ref[1:2, :]
    con_a = rows_ref[2:3, :]
    con_b = rows_ref[3:4, :]
    con_c = rows_ref[4:5, :]
    op_row = rows_ref[5:6, :]
    validf = rows_ref[10:11, :]

    dx = px - xf
    dy = py - yf
    power = -0.5 * (con_a * dx * dx + con_c * dy * dy) - con_b * dx * dy
    alpha = jnp.minimum(0.99, op_row * jnp.exp(power))
    alpha = jnp.where(power > 0.0, 0.0, alpha)
    alpha = jnp.where(alpha < (1.0 / 255.0), 0.0, alpha)
    alpha = alpha * validf

    lane = lax.broadcasted_iota(jnp.int32, (1, P), 1)
    s = 1.0 - alpha
    k = 1
    while k < P:
        rolled = jnp.roll(s, k, axis=1)
        s = s * jnp.where(lane < k, 1.0, rolled)
        k *= 2
    r1 = jnp.roll(s, 1, axis=1)
    cp_last = r1[:, 0:1]
    excl = jnp.where(lane < 1, 1.0, r1)
    wgt = excl * alpha

    cmat3 = cols_ref[:, 6:9].astype(jnp.bfloat16)
    rgb = lax.dot_general(wgt.astype(jnp.bfloat16), cmat3,
                          (((1,), (0,)), ((), ())),
                          preferred_element_type=jnp.float32)
    invc = cols_ref[:, 9:10]
    invd = lax.dot_general(wgt, invc, (((1,), (0,)), ((), ())),
                           preferred_element_type=jnp.float32,
                           precision=lax.Precision.HIGHEST)
    out_ref[...] = jnp.concatenate([rgb, invd], axis=1) + cp_last * bg_ref[0:1, :]


@jax.jit
def _run(g3, opr, colr, scr, rotr, vm, pm, bg4):
    rows, cols, radii = pl.pallas_call(
        _prep_kernel,
        out_shape=[
            jax.ShapeDtypeStruct((NROWS, P), jnp.float32),
            jax.ShapeDtypeStruct((P, NROWS), jnp.float32),
            jax.ShapeDtypeStruct((1, P), jnp.int32),
        ],
        in_specs=[
            pl.BlockSpec(memory_space=pltpu.VMEM),
            pl.BlockSpec(memory_space=pltpu.VMEM),
            pl.BlockSpec(memory_space=pltpu.VMEM),
            pl.BlockSpec(memory_space=pltpu.VMEM),
            pl.BlockSpec(memory_space=pltpu.VMEM),
            pl.BlockSpec(memory_space=pltpu.SMEM),
            pl.BlockSpec(memory_space=pltpu.SMEM),
        ],
    )(g3, opr, colr, scr, rotr, vm, pm)

    out = pl.pallas_call(
        _comp_kernel,
        grid=(HW // NPC,),
        out_shape=jax.ShapeDtypeStruct((HW, 4), jnp.float32),
        in_specs=[
            pl.BlockSpec((NROWS, P), lambda i: (0, 0)),
            pl.BlockSpec((P, NROWS), lambda i: (0, 0)),
            pl.BlockSpec((1, 4), lambda i: (0, 0)),
        ],
        out_specs=pl.BlockSpec((NPC, 4), lambda i: (i, 0)),
    )(rows, cols, bg4)
    return rows, cols, radii, out


def kernel(means3D, means2D, opacities, colors_precomp, scales, rotations,
           bg, viewmatrix, projmatrix, campos):
    g3 = means3D.T.astype(jnp.float32)
    opr = opacities.T.astype(jnp.float32)
    colr = colors_precomp.T.astype(jnp.float32)
    scr = scales.T.astype(jnp.float32)
    rotr = rotations.T.astype(jnp.float32)
    bg4 = jnp.concatenate([bg.astype(jnp.float32),
                           jnp.zeros((1,), jnp.float32)]).reshape(1, 4)
    vm_r = viewmatrix.astype(jnp.float32).astype(jnp.bfloat16).astype(jnp.float32)
    pm_r = projmatrix.astype(jnp.float32).astype(jnp.bfloat16).astype(jnp.float32)
    _, _, radii, out = _run(g3, opr, colr, scr, rotr, vm_r, pm_r, bg4)
    color = out[:, :3].T.reshape(3, H, W)
    invdepth = out[:, 3].reshape(1, H, W)
    return color, radii.reshape(P), invdepth
